# baseline (speedup 1.0000x reference)
import jax
import jax.numpy as jnp
from jax import lax
from jax.experimental import pallas as pl
from jax.experimental.pallas import tpu as pltpu

_R, _N = 128, 32768
_BR = 8
_BC = 4096
_NC = _N // _BC


def _argmax_body(x_ref, max_ref, idx_ref, mscr, iscr):
    j = pl.program_id(1)
    x = x_ref[...]
    m = jnp.max(x, axis=1, keepdims=True)
    iota = lax.broadcasted_iota(jnp.int32, x.shape, 1) + j * _BC
    first = jnp.min(jnp.where(x == m, iota, _N), axis=1, keepdims=True)

    @pl.when(j == 0)
    def _():
        mscr[...] = m
        iscr[...] = first

    @pl.when(j > 0)
    def _():
        mprev = mscr[...]
        better = m > mprev
        mscr[...] = jnp.where(better, m, mprev)
        iscr[...] = jnp.where(better, first, iscr[...])

    @pl.when(j == _NC - 1)
    def _():
        max_ref[...] = mscr[...]
        idx_ref[...] = iscr[...]


def _argmax_tc(probs):
    return pl.pallas_call(
        _argmax_body,
        grid=(_R // _BR, _NC),
        in_specs=[pl.BlockSpec((_BR, _BC), lambda i, j: (i, j))],
        out_specs=[
            pl.BlockSpec((_BR, 1), lambda i, j: (i, 0)),
            pl.BlockSpec((_BR, 1), lambda i, j: (i, 0)),
        ],
        out_shape=[
            jax.ShapeDtypeStruct((_R, 1), jnp.float32),
            jax.ShapeDtypeStruct((_R, 1), jnp.int32),
        ],
        scratch_shapes=[
            pltpu.VMEM((_BR, 1), jnp.float32),
            pltpu.VMEM((_BR, 1), jnp.int32),
        ],
    )(probs)


def kernel(probs):
    _, idx = _argmax_tc(probs)
    onehot = jnp.arange(_N, dtype=jnp.int32)[None, :] == idx
    return onehot


# speedup vs baseline: 3.6948x; 3.6948x over previous
import jax
import jax.numpy as jnp
from jax import lax
from jax.experimental import pallas as pl

_R, _N = 128, 32768
_BR = 8


def _argmax_body(x_ref, idx_ref):
    x = x_ref[...]
    m = jnp.max(x, axis=1, keepdims=True)
    iota = lax.broadcasted_iota(jnp.int32, x.shape, 1)
    idx_ref[...] = jnp.min(jnp.where(x == m, iota, _N), axis=1, keepdims=True)


def _argmax_tc(probs):
    return pl.pallas_call(
        _argmax_body,
        grid=(_R // _BR,),
        in_specs=[pl.BlockSpec((_BR, _N), lambda i: (i, 0))],
        out_specs=pl.BlockSpec((_BR, 1), lambda i: (i, 0)),
        out_shape=jax.ShapeDtypeStruct((_R, 1), jnp.int32),
    )(probs)


def kernel(probs):
    idx = _argmax_tc(probs)
    onehot = jnp.arange(_N, dtype=jnp.int32)[None, :] == idx
    return onehot


# speedup vs baseline: 4.9778x; 1.3472x over previous
import jax
import jax.numpy as jnp
from jax import lax
from jax.experimental import pallas as pl

_R, _N = 128, 32768
_BR = 16


def _argmax_body(x_ref, idx_ref):
    x = x_ref[...]
    m = jnp.max(x, axis=1, keepdims=True)
    iota = lax.broadcasted_iota(jnp.int32, x.shape, 1)
    idx_ref[...] = jnp.min(jnp.where(x == m, iota, _N), axis=1, keepdims=True)


def _argmax_tc(probs):
    return pl.pallas_call(
        _argmax_body,
        grid=(_R // _BR,),
        in_specs=[pl.BlockSpec((_BR, _N), lambda i: (i, 0))],
        out_specs=pl.BlockSpec((_BR, 1), lambda i: (i, 0)),
        out_shape=jax.ShapeDtypeStruct((_R, 1), jnp.int32),
    )(probs)


def kernel(probs):
    idx = _argmax_tc(probs)
    onehot = jnp.arange(_N, dtype=jnp.int32)[None, :] == idx
    return onehot


# speedup vs baseline: 5.8211x; 1.1694x over previous
import jax
import jax.numpy as jnp
from jax import lax
from jax.experimental import pallas as pl

_R, _N = 128, 32768
_BR = 32


def _argmax_body(x_ref, idx_ref):
    x = x_ref[...]
    m = jnp.max(x, axis=1, keepdims=True)
    iota = lax.broadcasted_iota(jnp.int32, x.shape, 1)
    idx_ref[...] = jnp.min(jnp.where(x == m, iota, _N), axis=1, keepdims=True)


def _argmax_tc(probs):
    return pl.pallas_call(
        _argmax_body,
        grid=(_R // _BR,),
        in_specs=[pl.BlockSpec((_BR, _N), lambda i: (i, 0))],
        out_specs=pl.BlockSpec((_BR, 1), lambda i: (i, 0)),
        out_shape=jax.ShapeDtypeStruct((_R, 1), jnp.int32),
    )(probs)


def kernel(probs):
    idx = _argmax_tc(probs)
    onehot = jnp.arange(_N, dtype=jnp.int32)[None, :] == idx
    return onehot


# speedup vs baseline: 6.0711x; 1.0429x over previous
import jax
import jax.numpy as jnp
from jax import lax
from jax.experimental import pallas as pl

_R, _N = 128, 32768
_BR = 64


def _argmax_body(x_ref, idx_ref):
    x = x_ref[...]
    m = jnp.max(x, axis=1, keepdims=True)
    iota = lax.broadcasted_iota(jnp.int32, x.shape, 1)
    idx_ref[...] = jnp.min(jnp.where(x == m, iota, _N), axis=1, keepdims=True)


def _argmax_tc(probs):
    return pl.pallas_call(
        _argmax_body,
        grid=(_R // _BR,),
        in_specs=[pl.BlockSpec((_BR, _N), lambda i: (i, 0))],
        out_specs=pl.BlockSpec((_BR, 1), lambda i: (i, 0)),
        out_shape=jax.ShapeDtypeStruct((_R, 1), jnp.int32),
    )(probs)


def kernel(probs):
    idx = _argmax_tc(probs)
    onehot = jnp.arange(_N, dtype=jnp.int32)[None, :] == idx
    return onehot
